# baseline (speedup 1.0000x reference)
import functools

import jax
import jax.numpy as jnp
from jax import lax
from jax.experimental import pallas as pl
from jax.experimental.pallas import tpu as pltpu
from jax.experimental.pallas import tpu_sc as plsc

N = 10000
E = 320000
IN_CH = 128
HIDDEN = 64
NUM_GRAPHS = 64

NC = 2
NS = 16
NW = NC * NS
EPW = E // NW
CH = 40
NCHUNK = EPW // CH
RPT = N // NS
DEGW = 16

_mesh = plsc.VectorSubcoreMesh(core_axis_name="c", subcore_axis_name="s")
_sc_params = pltpu.CompilerParams(use_tc_tiling_on_sc=False)


NBUF = 10
DEGG = 5


@functools.partial(
    pl.kernel,
    out_type=jax.ShapeDtypeStruct((NC, N, DEGW), jnp.float32),
    mesh=_mesh,
    compiler_params=_sc_params,
    scratch_types=[
        pltpu.VMEM((NCHUNK, CH), jnp.int32),
        pltpu.VMEM((CH, DEGW), jnp.float32),
        pltpu.VMEM_SHARED((N, DEGW), jnp.float32),
        pltpu.SemaphoreType.DMA,
    ],
)
def _sc_degree(dst_hbm, ones_hbm, zeros_hbm, out_hbm, dst_v, ones_v, acc_sh,
               sem):
    cid = lax.axis_index("c")
    sid = lax.axis_index("s")
    wid = cid * NS + sid
    pltpu.sync_copy(dst_hbm.at[wid], dst_v)
    pltpu.sync_copy(ones_hbm, ones_v)
    pltpu.sync_copy(zeros_hbm, acc_sh.at[pl.ds(sid * RPT, RPT)])
    plsc.subcore_barrier()

    @pl.loop(0, NCHUNK // DEGG)
    def _(j):
        cps = [pltpu.async_copy(ones_v, acc_sh.at[dst_v.at[j * DEGG + g]],
                                sem, add=True)
               for g in range(DEGG)]
        for cp in cps:
            cp.wait()

    plsc.subcore_barrier()
    pltpu.sync_copy(acc_sh.at[pl.ds(sid * RPT, RPT)],
                    out_hbm.at[cid, pl.ds(sid * RPT, RPT)])


@functools.partial(
    pl.kernel,
    out_type=jax.ShapeDtypeStruct((NC, N, HIDDEN), jnp.float32),
    mesh=_mesh,
    compiler_params=_sc_params,
    scratch_types=[
        pltpu.VMEM((NCHUNK, CH), jnp.int32),
        pltpu.VMEM((NCHUNK, CH), jnp.int32),
        [pltpu.VMEM((CH, HIDDEN), jnp.float32)] * NBUF,
        pltpu.VMEM_SHARED((N, HIDDEN), jnp.float32),
        [pltpu.SemaphoreType.DMA] * NBUF,
        [pltpu.SemaphoreType.DMA] * NBUF,
    ],
)
def _sc_scatter(y_hbm, src_hbm, dst_hbm, zeros_hbm, out_hbm,
                src_v, dst_v, rows_v, acc_sh, gsems, ssems):
    cid = lax.axis_index("c")
    sid = lax.axis_index("s")
    wid = cid * NS + sid
    pltpu.sync_copy(src_hbm.at[wid], src_v)
    pltpu.sync_copy(dst_hbm.at[wid], dst_v)
    pltpu.sync_copy(zeros_hbm, acc_sh.at[pl.ds(sid * RPT, RPT)])
    plsc.subcore_barrier()

    for b in range(NBUF):
        pltpu.async_copy(y_hbm.at[src_v.at[b]], rows_v[b], gsems[b])

    @pl.loop(0, NCHUNK // NBUF - 1)
    def _(j):
        i0 = j * NBUF
        for b in range(NBUF):
            pltpu.make_async_copy(y_hbm.at[src_v.at[i0 + b]], rows_v[b],
                                  gsems[b]).wait()
            pltpu.async_copy(rows_v[b], acc_sh.at[dst_v.at[i0 + b]],
                             ssems[b], add=True)
        for b in range(NBUF):
            pltpu.make_async_copy(rows_v[b], acc_sh.at[dst_v.at[i0 + b]],
                                  ssems[b]).wait()
            pltpu.async_copy(y_hbm.at[src_v.at[i0 + NBUF + b]], rows_v[b],
                             gsems[b])

    last = NCHUNK - NBUF
    for b in range(NBUF):
        pltpu.make_async_copy(y_hbm.at[src_v.at[last + b]], rows_v[b],
                              gsems[b]).wait()
        pltpu.async_copy(rows_v[b], acc_sh.at[dst_v.at[last + b]],
                         ssems[b], add=True)
    for b in range(NBUF):
        pltpu.make_async_copy(rows_v[b], acc_sh.at[dst_v.at[last + b]],
                              ssems[b]).wait()

    plsc.subcore_barrier()
    pltpu.sync_copy(acc_sh.at[pl.ds(sid * RPT, RPT)],
                    out_hbm.at[cid, pl.ds(sid * RPT, RPT)])


def _tc_mm_scale_body(x_ref, w_ref, dp_ref, y_ref, dinv_ref):
    deg = dp_ref[0, :, 0:1] + dp_ref[1, :, 0:1] + 1.0
    dinv = lax.rsqrt(jnp.maximum(deg, 1.0))
    dinv_ref[...] = dinv
    y_ref[...] = jnp.dot(x_ref[...], w_ref[...],
                         preferred_element_type=jnp.float32) * dinv


def _tc_mid_body(z_ref, y1_ref, dinv_ref, b_ref, w_ref, y2_ref):
    dinv = dinv_ref[...]
    h = jnp.maximum(
        (z_ref[0] + z_ref[1] + y1_ref[...]) * dinv + b_ref[...], 0.0)
    y2_ref[...] = jnp.dot(h, w_ref[...],
                          preferred_element_type=jnp.float32) * dinv


def _tc_final_body(z_ref, y2_ref, dinv_ref, b_ref, batch_ref,
                   wh_ref, bh_ref, out_ref):
    h = jnp.maximum(
        (z_ref[0] + z_ref[1] + y2_ref[...]) * dinv_ref[...]
        + b_ref[...], 0.0)
    gids = lax.broadcasted_iota(jnp.int32, (NUM_GRAPHS, N), 0)
    oh = (batch_ref[...] == gids).astype(jnp.float32)
    sums = jnp.dot(oh, h, preferred_element_type=jnp.float32)
    counts = jnp.sum(oh, axis=1, keepdims=True)
    pooled = sums / jnp.maximum(counts, 1.0)
    out_ref[...] = jnp.dot(pooled, wh_ref[...],
                           preferred_element_type=jnp.float32) + bh_ref[...]


def kernel(x, edge_index, batch, W1, b1, W2, b2, Wh, bh):
    src = edge_index[0].astype(jnp.int32).reshape(NW, NCHUNK, CH)
    dst = edge_index[1].astype(jnp.int32).reshape(NW, NCHUNK, CH)
    batch2 = batch.astype(jnp.int32).reshape(1, N)
    ones_deg = jnp.ones((CH, DEGW), jnp.float32)
    zeros_deg = jnp.zeros((RPT, DEGW), jnp.float32)
    zeros_acc = jnp.zeros((RPT, HIDDEN), jnp.float32)

    degp = _sc_degree(dst, ones_deg, zeros_deg)
    y1, dinv = pl.pallas_call(
        _tc_mm_scale_body,
        out_shape=(jax.ShapeDtypeStruct((N, HIDDEN), jnp.float32),
                   jax.ShapeDtypeStruct((N, 1), jnp.float32)),
    )(x, W1, degp)

    z1 = _sc_scatter(y1, src, dst, zeros_acc)
    y2 = pl.pallas_call(
        _tc_mid_body,
        out_shape=jax.ShapeDtypeStruct((N, HIDDEN), jnp.float32),
    )(z1, y1, dinv, b1.reshape(1, HIDDEN), W2)

    z2 = _sc_scatter(y2, src, dst, zeros_acc)
    out = pl.pallas_call(
        _tc_final_body,
        out_shape=jax.ShapeDtypeStruct((NUM_GRAPHS, 2), jnp.float32),
    )(z2, y2, dinv, b2.reshape(1, HIDDEN), batch2,
      Wh, bh.reshape(1, 2))
    return out


# speedup vs baseline: 1.0376x; 1.0376x over previous
import functools

import jax
import jax.numpy as jnp
from jax import lax
from jax.experimental import pallas as pl
from jax.experimental.pallas import tpu as pltpu
from jax.experimental.pallas import tpu_sc as plsc

N = 10000
E = 320000
IN_CH = 128
HIDDEN = 64
NUM_GRAPHS = 64

NC = 2
NS = 16
NW = NC * NS
EPW = E // NW
CH = 80
NCHUNK = EPW // CH
RPT = N // NS
DEGW = 16

_mesh = plsc.VectorSubcoreMesh(core_axis_name="c", subcore_axis_name="s")
_sc_params = pltpu.CompilerParams(use_tc_tiling_on_sc=False)


NBUF = 5
DEGG = 5


@functools.partial(
    pl.kernel,
    out_type=jax.ShapeDtypeStruct((NC, N, DEGW), jnp.float32),
    mesh=_mesh,
    compiler_params=_sc_params,
    scratch_types=[
        pltpu.VMEM((NCHUNK, CH), jnp.int32),
        pltpu.VMEM((CH, DEGW), jnp.float32),
        pltpu.VMEM_SHARED((N, DEGW), jnp.float32),
        pltpu.SemaphoreType.DMA,
    ],
)
def _sc_degree(dst_hbm, ones_hbm, zeros_hbm, out_hbm, dst_v, ones_v, acc_sh,
               sem):
    cid = lax.axis_index("c")
    sid = lax.axis_index("s")
    wid = cid * NS + sid
    pltpu.sync_copy(dst_hbm.at[wid], dst_v)
    pltpu.sync_copy(ones_hbm, ones_v)
    pltpu.sync_copy(zeros_hbm, acc_sh.at[pl.ds(sid * RPT, RPT)])
    plsc.subcore_barrier()

    @pl.loop(0, NCHUNK // DEGG)
    def _(j):
        cps = [pltpu.async_copy(ones_v, acc_sh.at[dst_v.at[j * DEGG + g]],
                                sem, add=True)
               for g in range(DEGG)]
        for cp in cps:
            cp.wait()

    plsc.subcore_barrier()
    pltpu.sync_copy(acc_sh.at[pl.ds(sid * RPT, RPT)],
                    out_hbm.at[cid, pl.ds(sid * RPT, RPT)])


@functools.partial(
    pl.kernel,
    out_type=jax.ShapeDtypeStruct((NC, N, HIDDEN), jnp.float32),
    mesh=_mesh,
    compiler_params=_sc_params,
    scratch_types=[
        pltpu.VMEM((NCHUNK, CH), jnp.int32),
        pltpu.VMEM((NCHUNK, CH), jnp.int32),
        [pltpu.VMEM((CH, HIDDEN), jnp.float32)] * NBUF,
        pltpu.VMEM_SHARED((N, HIDDEN), jnp.float32),
        [pltpu.SemaphoreType.DMA] * NBUF,
        [pltpu.SemaphoreType.DMA] * NBUF,
    ],
)
def _sc_scatter(y_hbm, src_hbm, dst_hbm, zeros_hbm, out_hbm,
                src_v, dst_v, rows_v, acc_sh, gsems, ssems):
    cid = lax.axis_index("c")
    sid = lax.axis_index("s")
    wid = cid * NS + sid
    pltpu.sync_copy(src_hbm.at[wid], src_v)
    pltpu.sync_copy(dst_hbm.at[wid], dst_v)
    pltpu.sync_copy(zeros_hbm, acc_sh.at[pl.ds(sid * RPT, RPT)])
    plsc.subcore_barrier()

    for b in range(NBUF):
        pltpu.async_copy(y_hbm.at[src_v.at[b]], rows_v[b], gsems[b])

    @pl.loop(0, NCHUNK // NBUF - 1)
    def _(j):
        i0 = j * NBUF
        for b in range(NBUF):
            pltpu.make_async_copy(y_hbm.at[src_v.at[i0 + b]], rows_v[b],
                                  gsems[b]).wait()
            pltpu.async_copy(rows_v[b], acc_sh.at[dst_v.at[i0 + b]],
                             ssems[b], add=True)
        for b in range(NBUF):
            pltpu.make_async_copy(rows_v[b], acc_sh.at[dst_v.at[i0 + b]],
                                  ssems[b]).wait()
            pltpu.async_copy(y_hbm.at[src_v.at[i0 + NBUF + b]], rows_v[b],
                             gsems[b])

    last = NCHUNK - NBUF
    for b in range(NBUF):
        pltpu.make_async_copy(y_hbm.at[src_v.at[last + b]], rows_v[b],
                              gsems[b]).wait()
        pltpu.async_copy(rows_v[b], acc_sh.at[dst_v.at[last + b]],
                         ssems[b], add=True)
    for b in range(NBUF):
        pltpu.make_async_copy(rows_v[b], acc_sh.at[dst_v.at[last + b]],
                              ssems[b]).wait()

    plsc.subcore_barrier()
    pltpu.sync_copy(acc_sh.at[pl.ds(sid * RPT, RPT)],
                    out_hbm.at[cid, pl.ds(sid * RPT, RPT)])


def _tc_mm_scale_body(x_ref, w_ref, dp_ref, y_ref, dinv_ref):
    deg = dp_ref[0, :, 0:1] + dp_ref[1, :, 0:1] + 1.0
    dinv = lax.rsqrt(jnp.maximum(deg, 1.0))
    dinv_ref[...] = dinv
    y_ref[...] = jnp.dot(x_ref[...], w_ref[...],
                         preferred_element_type=jnp.float32) * dinv


def _tc_mid_body(z_ref, y1_ref, dinv_ref, b_ref, w_ref, y2_ref):
    dinv = dinv_ref[...]
    h = jnp.maximum(
        (z_ref[0] + z_ref[1] + y1_ref[...]) * dinv + b_ref[...], 0.0)
    y2_ref[...] = jnp.dot(h, w_ref[...],
                          preferred_element_type=jnp.float32) * dinv


def _tc_final_body(z_ref, y2_ref, dinv_ref, b_ref, batch_ref,
                   wh_ref, bh_ref, out_ref):
    h = jnp.maximum(
        (z_ref[0] + z_ref[1] + y2_ref[...]) * dinv_ref[...]
        + b_ref[...], 0.0)
    gids = lax.broadcasted_iota(jnp.int32, (NUM_GRAPHS, N), 0)
    oh = (batch_ref[...] == gids).astype(jnp.float32)
    sums = jnp.dot(oh, h, preferred_element_type=jnp.float32)
    counts = jnp.sum(oh, axis=1, keepdims=True)
    pooled = sums / jnp.maximum(counts, 1.0)
    out_ref[...] = jnp.dot(pooled, wh_ref[...],
                           preferred_element_type=jnp.float32) + bh_ref[...]


def kernel(x, edge_index, batch, W1, b1, W2, b2, Wh, bh):
    src = edge_index[0].astype(jnp.int32).reshape(NW, NCHUNK, CH)
    dst = edge_index[1].astype(jnp.int32).reshape(NW, NCHUNK, CH)
    batch2 = batch.astype(jnp.int32).reshape(1, N)
    ones_deg = jnp.ones((CH, DEGW), jnp.float32)
    zeros_deg = jnp.zeros((RPT, DEGW), jnp.float32)
    zeros_acc = jnp.zeros((RPT, HIDDEN), jnp.float32)

    degp = _sc_degree(dst, ones_deg, zeros_deg)
    y1, dinv = pl.pallas_call(
        _tc_mm_scale_body,
        out_shape=(jax.ShapeDtypeStruct((N, HIDDEN), jnp.float32),
                   jax.ShapeDtypeStruct((N, 1), jnp.float32)),
    )(x, W1, degp)

    z1 = _sc_scatter(y1, src, dst, zeros_acc)
    y2 = pl.pallas_call(
        _tc_mid_body,
        out_shape=jax.ShapeDtypeStruct((N, HIDDEN), jnp.float32),
    )(z1, y1, dinv, b1.reshape(1, HIDDEN), W2)

    z2 = _sc_scatter(y2, src, dst, zeros_acc)
    out = pl.pallas_call(
        _tc_final_body,
        out_shape=jax.ShapeDtypeStruct((NUM_GRAPHS, 2), jnp.float32),
    )(z2, y2, dinv, b2.reshape(1, HIDDEN), batch2,
      Wh, bh.reshape(1, 2))
    return out


# speedup vs baseline: 1.1471x; 1.1056x over previous
import functools

import jax
import jax.numpy as jnp
from jax import lax
from jax.experimental import pallas as pl
from jax.experimental.pallas import tpu as pltpu
from jax.experimental.pallas import tpu_sc as plsc

N = 10000
E = 320000
IN_CH = 128
HIDDEN = 64
NUM_GRAPHS = 64

NC = 2
NS = 16
NW = NC * NS
EPW = E // NW
CH = 80
NCHUNK = EPW // CH
RPT = N // NS
DEGW = 16

_mesh = plsc.VectorSubcoreMesh(core_axis_name="c", subcore_axis_name="s")
_sc_params = pltpu.CompilerParams(use_tc_tiling_on_sc=False)


NBUF = 5
DEGG = 5


@functools.partial(
    pl.kernel,
    out_type=jax.ShapeDtypeStruct((NC, N, DEGW), jnp.float32),
    mesh=_mesh,
    compiler_params=_sc_params,
    scratch_types=[
        pltpu.VMEM((NCHUNK, CH), jnp.int32),
        pltpu.VMEM((CH, DEGW), jnp.float32),
        pltpu.VMEM_SHARED((N, DEGW), jnp.float32),
        pltpu.SemaphoreType.DMA,
    ],
)
def _sc_degree(dst_hbm, ones_hbm, zeros_hbm, out_hbm, dst_v, ones_v, acc_sh,
               sem):
    cid = lax.axis_index("c")
    sid = lax.axis_index("s")
    wid = cid * NS + sid
    cps = [pltpu.async_copy(dst_hbm.at[wid], dst_v, sem),
           pltpu.async_copy(ones_hbm, ones_v, sem),
           pltpu.async_copy(zeros_hbm, acc_sh.at[pl.ds(sid * RPT, RPT)],
                            sem)]
    for cp in cps:
        cp.wait()
    plsc.subcore_barrier()

    @pl.loop(0, NCHUNK // DEGG)
    def _(j):
        cps = [pltpu.async_copy(ones_v, acc_sh.at[dst_v.at[j * DEGG + g]],
                                sem, add=True)
               for g in range(DEGG)]
        for cp in cps:
            cp.wait()

    plsc.subcore_barrier()
    pltpu.sync_copy(acc_sh.at[pl.ds(sid * RPT, RPT)],
                    out_hbm.at[cid, pl.ds(sid * RPT, RPT)])


@functools.partial(
    pl.kernel,
    out_type=jax.ShapeDtypeStruct((NC, N, HIDDEN), jnp.float32),
    mesh=_mesh,
    compiler_params=_sc_params,
    scratch_types=[
        pltpu.VMEM((NCHUNK, CH), jnp.int32),
        pltpu.VMEM((NCHUNK, CH), jnp.int32),
        [pltpu.VMEM((CH, HIDDEN), jnp.float32)] * NBUF,
        pltpu.VMEM_SHARED((N, HIDDEN), jnp.float32),
        [pltpu.SemaphoreType.DMA] * NBUF,
    ],
)
def _sc_scatter(y_hbm, src_hbm, dst_hbm, zeros_hbm, out_hbm,
                src_v, dst_v, rows_v, acc_sh, sems):
    cid = lax.axis_index("c")
    sid = lax.axis_index("s")
    wid = cid * NS + sid
    cps = [pltpu.async_copy(src_hbm.at[wid], src_v, sems[0]),
           pltpu.async_copy(dst_hbm.at[wid], dst_v, sems[1]),
           pltpu.async_copy(zeros_hbm, acc_sh.at[pl.ds(sid * RPT, RPT)],
                            sems[2])]
    for cp in cps:
        cp.wait()
    plsc.subcore_barrier()

    for b in range(NBUF):
        pltpu.async_copy(y_hbm.at[src_v.at[b]], rows_v[b], sems[b])

    @pl.loop(0, NCHUNK // NBUF - 1)
    def _(j):
        i0 = j * NBUF
        for b in range(NBUF):
            pltpu.make_async_copy(y_hbm.at[src_v.at[i0 + b]], rows_v[b],
                                  sems[b]).wait()
            pltpu.sync_copy(rows_v[b], acc_sh.at[dst_v.at[i0 + b]], add=True)
            pltpu.async_copy(y_hbm.at[src_v.at[i0 + NBUF + b]], rows_v[b],
                             sems[b])

    last = NCHUNK - NBUF
    for b in range(NBUF):
        pltpu.make_async_copy(y_hbm.at[src_v.at[last + b]], rows_v[b],
                              sems[b]).wait()
        pltpu.sync_copy(rows_v[b], acc_sh.at[dst_v.at[last + b]], add=True)

    plsc.subcore_barrier()
    pltpu.sync_copy(acc_sh.at[pl.ds(sid * RPT, RPT)],
                    out_hbm.at[cid, pl.ds(sid * RPT, RPT)])


def _tc_mm_scale_body(x_ref, w_ref, dp_ref, y_ref, dinv_ref):
    deg = dp_ref[0, :, 0:1] + dp_ref[1, :, 0:1] + 1.0
    dinv = lax.rsqrt(jnp.maximum(deg, 1.0))
    dinv_ref[...] = dinv
    y_ref[...] = jnp.dot(x_ref[...], w_ref[...],
                         preferred_element_type=jnp.float32) * dinv


def _tc_mid_body(z_ref, y1_ref, dinv_ref, b_ref, w_ref, y2_ref):
    dinv = dinv_ref[...]
    h = jnp.maximum(
        (z_ref[0] + z_ref[1] + y1_ref[...]) * dinv + b_ref[...], 0.0)
    y2_ref[...] = jnp.dot(h, w_ref[...],
                          preferred_element_type=jnp.float32) * dinv


def _tc_final_body(z_ref, y2_ref, dinv_ref, b_ref, batch_ref,
                   wh_ref, bh_ref, out_ref):
    h = jnp.maximum(
        (z_ref[0] + z_ref[1] + y2_ref[...]) * dinv_ref[...]
        + b_ref[...], 0.0)
    gids = lax.broadcasted_iota(jnp.int32, (NUM_GRAPHS, N), 0)
    oh = (batch_ref[...] == gids).astype(jnp.float32)
    sums = jnp.dot(oh, h, preferred_element_type=jnp.float32)
    counts = jnp.sum(oh, axis=1, keepdims=True)
    pooled = sums / jnp.maximum(counts, 1.0)
    out_ref[...] = jnp.dot(pooled, wh_ref[...],
                           preferred_element_type=jnp.float32) + bh_ref[...]


def kernel(x, edge_index, batch, W1, b1, W2, b2, Wh, bh):
    src = edge_index[0].astype(jnp.int32).reshape(NW, NCHUNK, CH)
    dst = edge_index[1].astype(jnp.int32).reshape(NW, NCHUNK, CH)
    batch2 = batch.astype(jnp.int32).reshape(1, N)
    ones_deg = jnp.ones((CH, DEGW), jnp.float32)
    zeros_deg = jnp.zeros((RPT, DEGW), jnp.float32)
    zeros_acc = jnp.zeros((RPT, HIDDEN), jnp.float32)

    degp = _sc_degree(dst, ones_deg, zeros_deg)
    y1, dinv = pl.pallas_call(
        _tc_mm_scale_body,
        out_shape=(jax.ShapeDtypeStruct((N, HIDDEN), jnp.float32),
                   jax.ShapeDtypeStruct((N, 1), jnp.float32)),
    )(x, W1, degp)

    z1 = _sc_scatter(y1, src, dst, zeros_acc)
    y2 = pl.pallas_call(
        _tc_mid_body,
        out_shape=jax.ShapeDtypeStruct((N, HIDDEN), jnp.float32),
    )(z1, y1, dinv, b1.reshape(1, HIDDEN), W2)

    z2 = _sc_scatter(y2, src, dst, zeros_acc)
    out = pl.pallas_call(
        _tc_final_body,
        out_shape=jax.ShapeDtypeStruct((NUM_GRAPHS, 2), jnp.float32),
    )(z2, y2, dinv, b2.reshape(1, HIDDEN), batch2,
      Wh, bh.reshape(1, 2))
    return out


# speedup vs baseline: 1.1485x; 1.0012x over previous
import functools

import jax
import jax.numpy as jnp
from jax import lax
from jax.experimental import pallas as pl
from jax.experimental.pallas import tpu as pltpu
from jax.experimental.pallas import tpu_sc as plsc

N = 10000
E = 320000
IN_CH = 128
HIDDEN = 64
NUM_GRAPHS = 64

NC = 2
NS = 16
NW = NC * NS
EPW = E // NW
CH = 80
NCHUNK = EPW // CH
RPT = N // NS
DEGW = 16

_mesh = plsc.VectorSubcoreMesh(core_axis_name="c", subcore_axis_name="s")
_sc_params = pltpu.CompilerParams(use_tc_tiling_on_sc=False)


NBUF = 5
DEGG = 25


@functools.partial(
    pl.kernel,
    out_type=jax.ShapeDtypeStruct((NC, N, DEGW), jnp.float32),
    mesh=_mesh,
    compiler_params=_sc_params,
    scratch_types=[
        pltpu.VMEM((NCHUNK, CH), jnp.int32),
        pltpu.VMEM((CH, DEGW), jnp.float32),
        pltpu.VMEM_SHARED((N, DEGW), jnp.float32),
        pltpu.SemaphoreType.DMA,
    ],
)
def _sc_degree(dst_hbm, ones_hbm, zeros_hbm, out_hbm, dst_v, ones_v, acc_sh,
               sem):
    cid = lax.axis_index("c")
    sid = lax.axis_index("s")
    wid = cid * NS + sid
    cps = [pltpu.async_copy(dst_hbm.at[wid], dst_v, sem),
           pltpu.async_copy(ones_hbm, ones_v, sem),
           pltpu.async_copy(zeros_hbm, acc_sh.at[pl.ds(sid * RPT, RPT)],
                            sem)]
    for cp in cps:
        cp.wait()
    plsc.subcore_barrier()

    @pl.loop(0, NCHUNK // DEGG)
    def _(j):
        cps = [pltpu.async_copy(ones_v, acc_sh.at[dst_v.at[j * DEGG + g]],
                                sem, add=True)
               for g in range(DEGG)]
        for cp in cps:
            cp.wait()

    plsc.subcore_barrier()
    pltpu.sync_copy(acc_sh.at[pl.ds(sid * RPT, RPT)],
                    out_hbm.at[cid, pl.ds(sid * RPT, RPT)])


@functools.partial(
    pl.kernel,
    out_type=jax.ShapeDtypeStruct((NC, N, HIDDEN), jnp.float32),
    mesh=_mesh,
    compiler_params=_sc_params,
    scratch_types=[
        pltpu.VMEM((NCHUNK, CH), jnp.int32),
        pltpu.VMEM((NCHUNK, CH), jnp.int32),
        [pltpu.VMEM((CH, HIDDEN), jnp.float32)] * NBUF,
        pltpu.VMEM_SHARED((N, HIDDEN), jnp.float32),
        [pltpu.SemaphoreType.DMA] * NBUF,
    ],
)
def _sc_scatter(y_hbm, src_hbm, dst_hbm, zeros_hbm, out_hbm,
                src_v, dst_v, rows_v, acc_sh, sems):
    cid = lax.axis_index("c")
    sid = lax.axis_index("s")
    wid = cid * NS + sid
    cps = [pltpu.async_copy(src_hbm.at[wid], src_v, sems[0]),
           pltpu.async_copy(dst_hbm.at[wid], dst_v, sems[1]),
           pltpu.async_copy(zeros_hbm, acc_sh.at[pl.ds(sid * RPT, RPT)],
                            sems[2])]
    for cp in cps:
        cp.wait()
    plsc.subcore_barrier()

    for b in range(NBUF):
        pltpu.async_copy(y_hbm.at[src_v.at[b]], rows_v[b], sems[b])

    @pl.loop(0, NCHUNK // NBUF - 1)
    def _(j):
        i0 = j * NBUF
        for b in range(NBUF):
            pltpu.make_async_copy(y_hbm.at[src_v.at[i0 + b]], rows_v[b],
                                  sems[b]).wait()
            pltpu.sync_copy(rows_v[b], acc_sh.at[dst_v.at[i0 + b]], add=True)
            pltpu.async_copy(y_hbm.at[src_v.at[i0 + NBUF + b]], rows_v[b],
                             sems[b])

    last = NCHUNK - NBUF
    for b in range(NBUF):
        pltpu.make_async_copy(y_hbm.at[src_v.at[last + b]], rows_v[b],
                              sems[b]).wait()
        pltpu.sync_copy(rows_v[b], acc_sh.at[dst_v.at[last + b]], add=True)

    plsc.subcore_barrier()
    pltpu.sync_copy(acc_sh.at[pl.ds(sid * RPT, RPT)],
                    out_hbm.at[cid, pl.ds(sid * RPT, RPT)])


def _tc_mm_scale_body(x_ref, w_ref, dp_ref, y_ref, dinv_ref):
    deg = dp_ref[0, :, 0:1] + dp_ref[1, :, 0:1] + 1.0
    dinv = lax.rsqrt(jnp.maximum(deg, 1.0))
    dinv_ref[...] = dinv
    y_ref[...] = jnp.dot(x_ref[...], w_ref[...],
                         preferred_element_type=jnp.float32) * dinv


def _tc_mid_body(z_ref, y1_ref, dinv_ref, b_ref, w_ref, y2_ref):
    dinv = dinv_ref[...]
    h = jnp.maximum(
        (z_ref[0] + z_ref[1] + y1_ref[...]) * dinv + b_ref[...], 0.0)
    y2_ref[...] = jnp.dot(h, w_ref[...],
                          preferred_element_type=jnp.float32) * dinv


def _tc_final_body(z_ref, y2_ref, dinv_ref, b_ref, batch_ref,
                   wh_ref, bh_ref, out_ref):
    h = jnp.maximum(
        (z_ref[0] + z_ref[1] + y2_ref[...]) * dinv_ref[...]
        + b_ref[...], 0.0)
    gids = lax.broadcasted_iota(jnp.int32, (NUM_GRAPHS, N), 0)
    oh = (batch_ref[...] == gids).astype(jnp.float32)
    sums = jnp.dot(oh, h, preferred_element_type=jnp.float32)
    counts = jnp.sum(oh, axis=1, keepdims=True)
    pooled = sums / jnp.maximum(counts, 1.0)
    out_ref[...] = jnp.dot(pooled, wh_ref[...],
                           preferred_element_type=jnp.float32) + bh_ref[...]


def kernel(x, edge_index, batch, W1, b1, W2, b2, Wh, bh):
    src = edge_index[0].astype(jnp.int32).reshape(NW, NCHUNK, CH)
    dst = edge_index[1].astype(jnp.int32).reshape(NW, NCHUNK, CH)
    batch2 = batch.astype(jnp.int32).reshape(1, N)
    ones_deg = jnp.ones((CH, DEGW), jnp.float32)
    zeros_deg = jnp.zeros((RPT, DEGW), jnp.float32)
    zeros_acc = jnp.zeros((RPT, HIDDEN), jnp.float32)

    degp = _sc_degree(dst, ones_deg, zeros_deg)
    y1, dinv = pl.pallas_call(
        _tc_mm_scale_body,
        out_shape=(jax.ShapeDtypeStruct((N, HIDDEN), jnp.float32),
                   jax.ShapeDtypeStruct((N, 1), jnp.float32)),
    )(x, W1, degp)

    z1 = _sc_scatter(y1, src, dst, zeros_acc)
    y2 = pl.pallas_call(
        _tc_mid_body,
        out_shape=jax.ShapeDtypeStruct((N, HIDDEN), jnp.float32),
    )(z1, y1, dinv, b1.reshape(1, HIDDEN), W2)

    z2 = _sc_scatter(y2, src, dst, zeros_acc)
    out = pl.pallas_call(
        _tc_final_body,
        out_shape=jax.ShapeDtypeStruct((NUM_GRAPHS, 2), jnp.float32),
    )(z2, y2, dinv, b2.reshape(1, HIDDEN), batch2,
      Wh, bh.reshape(1, 2))
    return out
